# baseline (speedup 1.0000x reference)
import functools

import jax
import jax.numpy as jnp
from jax import lax
from jax.experimental import pallas as pl
from jax.experimental.pallas import tpu as pltpu
from jax.experimental.pallas import tpu_sc as plsc

_WIN = 128
_NBUF = 6


def _sc_gather(table, idx):
    T, D = table.shape
    E = idx.shape[0]
    nwin = E // _WIN
    mesh = plsc.VectorSubcoreMesh(core_axis_name="c", subcore_axis_name="s")
    NW = mesh.num_cores * mesh.num_subcores
    base, rem = divmod(nwin, NW)
    tmax = (base + 1 + _NBUF - 1) // _NBUF

    @functools.partial(
        pl.kernel,
        out_type=jax.ShapeDtypeStruct((E, D), table.dtype),
        mesh=mesh,
        scratch_types=[
            pltpu.VMEM((_NBUF, _WIN), jnp.int32),
            pltpu.VMEM((_NBUF, _WIN, D), table.dtype),
            pltpu.SemaphoreType.DMA((_NBUF,)),
            pltpu.SemaphoreType.DMA((_NBUF,)),
            pltpu.SemaphoreType.DMA((_NBUF,)),
        ],
    )
    def k(table_hbm, idx_hbm, out_hbm, idx_v, rows_v, sem_i, sem_g, sem_w):
        wid = lax.axis_index("s") * mesh.num_cores + lax.axis_index("c")
        lo = wid * base + jnp.minimum(wid, rem)
        hi = lo + base + jnp.where(wid < rem, 1, 0)

        def idx_copy(w, b):
            return pltpu.make_async_copy(
                idx_hbm.at[pl.ds(w * _WIN, _WIN)], idx_v.at[b], sem_i.at[b])

        def gather(b):
            return pltpu.make_async_copy(
                table_hbm.at[idx_v.at[b]], rows_v.at[b], sem_g.at[b])

        def writeout(w, b):
            return pltpu.make_async_copy(
                rows_v.at[b], out_hbm.at[pl.ds(w * _WIN, _WIN)], sem_w.at[b])

        for b in range(_NBUF):
            w = lo + b

            @pl.when(w < hi)
            def _():
                idx_copy(w, b).start()

        @pl.loop(0, tmax)
        def _(t):
            for b in range(_NBUF):
                w = lo + t * _NBUF + b

                @pl.when(w < hi)
                def _():
                    @pl.when(t > 0)
                    def _():
                        writeout(w, b).wait()

                    idx_copy(w, b).wait()
                    gather(b).start()

            for b in range(_NBUF):
                w = lo + t * _NBUF + b

                @pl.when(w < hi)
                def _():
                    gather(b).wait()
                    nw = w + _NBUF

                    @pl.when(nw < hi)
                    def _():
                        idx_copy(nw, b).start()

                    writeout(w, b).start()

        for b in range(_NBUF):
            writeout(lo, b).wait()

    return k(table, idx)


def _k1_body(atom_ref, r_ref):
    r_ref[...] = jnp.sqrt(jnp.abs(atom_ref[...]))


def _k3_body(nblk, m, g_ref, bond_ref, r_ref, wn_ref, bn_ref, au_ref):
    g = g_ref[...].reshape(nblk, m, g_ref.shape[-1])
    bq = bond_ref[...].reshape(nblk, m, bond_ref.shape[-1])
    inv = 1.0 / jnp.sum(bq * bq, axis=-1, keepdims=True)
    den = jnp.maximum(jnp.sum(inv, axis=1, keepdims=True), 1e-12)
    w3 = inv / den
    anw = jnp.sum(g * w3, axis=1)
    x = r_ref[...] * anw
    y = jnp.dot(x, wn_ref[...], preferred_element_type=jnp.float32) + bn_ref[...]
    au_ref[...] = jnp.maximum(y, 0.0).reshape(au_ref.shape)


def _k46_body(nbb, bond_ref, d0_ref, d1_ref, wt_ref, wb_ref, we_ref,
              bnte_ref, bedge_ref, out_ref, acc_ref):
    p = pl.program_id(0)
    i = pl.program_id(1)
    b = i // nbb

    @pl.when(p == 0)
    def _():
        @pl.when(i == 0)
        def _():
            acc_ref[...] = jnp.zeros_like(acc_ref)

        c0 = jnp.sum(d0_ref[...], axis=0, keepdims=True)
        c1 = jnp.sum(d1_ref[...], axis=0, keepdims=True)
        acc_ref[pl.ds(2 * b, 1), :] += c0
        acc_ref[pl.ds(2 * b + 1, 1), :] += c1

    @pl.when(p == 1)
    def _():
        s0 = acc_ref[pl.ds(2 * b, 1), :]
        s1 = acc_ref[pl.ds(2 * b + 1, 1), :]
        r0 = 1.0 / jnp.maximum(s0, 1e-12)
        r1 = 1.0 / jnp.maximum(s1, 1e-12)
        t = jnp.dot(d0_ref[...] * r0, wt_ref[...],
                    preferred_element_type=jnp.float32)
        t += jnp.dot(d1_ref[...] * r1, wb_ref[...],
                     preferred_element_type=jnp.float32)
        y = jnp.tanh(t + bnte_ref[...])
        fb = bond_ref.shape[-1]
        z = bond_ref[...].reshape(y.shape[0], fb) + y
        out_ref[...] = (
            jnp.dot(z, we_ref[...], preferred_element_type=jnp.float32)
            + bedge_ref[...]
        ).reshape(out_ref.shape)


def kernel(atom, bond, adj_matrix, adj_matrix_tuple, weight_node, weight_edge,
           weight_node_to_edge, bias_node, bias_edge, bias_node_to_edge):
    B, N, Fa = atom.shape
    M = adj_matrix.shape[-1]
    Fb = bond.shape[-1]
    BN = B * N
    NM = N * M
    TE = B * NM
    f32 = jnp.float32

    atom2 = atom.reshape(BN, Fa)

    blk1 = 2000
    R = pl.pallas_call(
        _k1_body,
        grid=(BN // blk1,),
        in_specs=[pl.BlockSpec((blk1, Fa), lambda i: (i, 0))],
        out_specs=pl.BlockSpec((blk1, Fa), lambda i: (i, 0)),
        out_shape=jax.ShapeDtypeStruct((BN, Fa), f32),
    )(atom2)

    offs = jnp.arange(B, dtype=jnp.int32) * N

    adjg = (adj_matrix + offs[:, None, None]).reshape(B * N * M)
    G = _sc_gather(R, adjg)

    blk3 = 400
    nb3 = N // blk3
    au3 = pl.pallas_call(
        functools.partial(_k3_body, blk3, M),
        grid=(BN // blk3,),
        in_specs=[
            pl.BlockSpec((blk3 * M, Fa), lambda i: (i, 0)),
            pl.BlockSpec((1, blk3, M, Fb),
                         lambda i: (i // nb3, i % nb3, 0, 0)),
            pl.BlockSpec((blk3, Fa), lambda i: (i, 0)),
            pl.BlockSpec((Fa, Fa), lambda i: (0, 0)),
            pl.BlockSpec((1, Fa), lambda i: (0, 0)),
        ],
        out_specs=pl.BlockSpec((1, blk3, Fa), lambda i: (i // nb3, i % nb3, 0)),
        out_shape=jax.ShapeDtypeStruct((B, N, Fa), f32),
    )(G, bond, R, weight_node, bias_node.reshape(1, Fa))
    au2 = au3.reshape(BN, Fa)

    I0 = adj_matrix_tuple[..., 0]
    I1 = adj_matrix_tuple[..., 1]
    I0g = (I0 + offs[:, None]).reshape(TE)
    I1g = (I1 + offs[:, None]).reshape(TE)
    D = _sc_gather(au2, jnp.concatenate([I0g, I1g]))

    blk6 = 2000
    nb6 = TE // blk6
    nbb = NM // blk6
    nrow6 = blk6 // M
    nr6 = N // nrow6
    outE = pl.pallas_call(
        functools.partial(_k46_body, nbb),
        grid=(2, nb6),
        in_specs=[
            pl.BlockSpec((1, nrow6, M, Fb),
                         lambda p, i: (i // nr6, i % nr6, 0, 0)),
            pl.BlockSpec((blk6, Fa), lambda p, i: (i, 0)),
            pl.BlockSpec((blk6, Fa), lambda p, i: (i + nb6, 0)),
            pl.BlockSpec((Fa, Fb), lambda p, i: (0, 0)),
            pl.BlockSpec((Fa, Fb), lambda p, i: (0, 0)),
            pl.BlockSpec((Fb, Fb), lambda p, i: (0, 0)),
            pl.BlockSpec((1, Fb), lambda p, i: (0, 0)),
            pl.BlockSpec((1, Fb), lambda p, i: (0, 0)),
        ],
        out_specs=pl.BlockSpec((1, nrow6, M, Fb),
                               lambda p, i: (i // nr6, i % nr6, 0, 0)),
        out_shape=jax.ShapeDtypeStruct((B, N, M, Fb), f32),
        scratch_shapes=[pltpu.VMEM((2 * B, Fa), f32)],
    )(bond, D, D,
      weight_node_to_edge[:Fa], weight_node_to_edge[Fa:], weight_edge,
      bias_node_to_edge.reshape(1, Fb), bias_edge.reshape(1, Fb))

    return (au3, outE)
